# Initial kernel scaffold; baseline (speedup 1.0000x reference)
import jax
import jax.numpy as jnp
from jax.experimental import pallas as pl


def kernel(h, edge_index):
    raise NotImplementedError("write your pallas kernel here")


# speedup vs baseline: 2.7715x; 2.7715x over previous
import functools

import jax
import jax.numpy as jnp
from jax import lax
from jax.experimental import pallas as pl
from jax.experimental.pallas import tpu as pltpu
from jax.experimental.pallas import tpu_sc as plsc

NC, NS, L = 2, 16, 16
NW = NC * NS
E = 320000
EPW = E // NW
CH = 80
NCH = EPW // CH
D = 128


def _dot_body(hb, src, dst, out, idx_s, idx_d, rows_s, rows_d, out_v, sem):
  wid = lax.axis_index("s") * NC + lax.axis_index("c")
  base = wid * EPW

  def chunk(g, carry):
    off = base + g * CH
    pltpu.sync_copy(src.at[pl.ds(off, CH)], idx_s)
    pltpu.sync_copy(dst.at[pl.ds(off, CH)], idx_d)
    pltpu.async_copy(hb.at[idx_s], rows_s, sem).wait()
    pltpu.async_copy(hb.at[idx_d], rows_d, sem).wait()

    lane = lax.iota(jnp.int32, L)
    last = lane == (L - 1)

    def edge(e, carry2):
      p = None
      for j in range(D // L):
        a = rows_s[e, pl.ds(j * L, L)]
        b = rows_d[e, pl.ds(j * L, L)]
        t = a * b
        p = t if p is None else p + t
      tot = plsc.cumsum(p)
      idx = jnp.full((L,), g * CH + e, jnp.int32)
      plsc.store_scatter(out_v, [idx], tot, mask=last)
      return carry2

    lax.fori_loop(0, CH, edge, 0, unroll=2)
    return carry

  lax.fori_loop(0, NCH, chunk, 0)
  pltpu.sync_copy(out_v, out.at[pl.ds(base, EPW)])


_dot_sc = functools.partial(
    pl.kernel,
    out_type=jax.ShapeDtypeStruct((E,), jnp.float32),
    mesh=plsc.VectorSubcoreMesh(
        core_axis_name="c", subcore_axis_name="s",
        num_cores=NC, num_subcores=NS),
    compiler_params=pltpu.CompilerParams(needs_layout_passes=False),
    scratch_types=[
        pltpu.VMEM((CH,), jnp.int32),
        pltpu.VMEM((CH,), jnp.int32),
        pltpu.VMEM((CH, D), jnp.float32),
        pltpu.VMEM((CH, D), jnp.float32),
        pltpu.VMEM((EPW,), jnp.float32),
        pltpu.SemaphoreType.DMA,
    ],
)(_dot_body)


@jax.jit
def kernel(h, edge_index):
  return _dot_sc(h, edge_index[0], edge_index[1])


# speedup vs baseline: 6.8614x; 2.4757x over previous
import functools

import jax
import jax.numpy as jnp
from jax import lax
from jax.experimental import pallas as pl
from jax.experimental.pallas import tpu as pltpu
from jax.experimental.pallas import tpu_sc as plsc

NC, NS, L = 2, 16, 16
NW = NC * NS
E = 320000
EPW = E // NW
CH = 80
NCH = EPW // CH
D = 128


def _dot_body(hb, src, dst, out, idx_s, idx_d, rows_s, rows_d, out_v, sems):
  wid = lax.axis_index("s") * NC + lax.axis_index("c")
  base = wid * EPW

  pltpu.sync_copy(src.at[pl.ds(base, EPW)], idx_s)
  pltpu.sync_copy(dst.at[pl.ds(base, EPW)], idx_d)

  def fire(g, b):
    isl = idx_s.at[pl.ds(g * CH, CH)]
    idl = idx_d.at[pl.ds(g * CH, CH)]
    pltpu.async_copy(hb.at[isl], rows_s.at[b], sems.at[2 * b])
    pltpu.async_copy(hb.at[idl], rows_d.at[b], sems.at[2 * b + 1])

  def drain(g, b):
    isl = idx_s.at[pl.ds(g * CH, CH)]
    idl = idx_d.at[pl.ds(g * CH, CH)]
    pltpu.make_async_copy(hb.at[isl], rows_s.at[b], sems.at[2 * b]).wait()
    pltpu.make_async_copy(hb.at[idl], rows_d.at[b], sems.at[2 * b + 1]).wait()

  lane = lax.iota(jnp.int32, L)
  last = lane == (L - 1)

  def compute(g, b):
    rs = rows_s.at[b]
    rd = rows_d.at[b]

    def edge(e, carry2):
      p = None
      for j in range(D // L):
        a = rs[e, pl.ds(j * L, L)]
        bb = rd[e, pl.ds(j * L, L)]
        t = a * bb
        p = t if p is None else p + t
      tot = plsc.cumsum(p)
      idx = jnp.full((L,), g * CH + e, jnp.int32)
      plsc.store_scatter(out_v, [idx], tot, mask=last)
      return carry2

    lax.fori_loop(0, CH, edge, 0, unroll=2)

  fire(0, 0)

  def step(gg, carry):
    for b in range(2):
      g = 2 * gg + b

      @pl.when(g + 1 < NCH)
      def _():
        fire(g + 1, 1 - b)

      drain(g, b)
      compute(g, b)
    return carry

  lax.fori_loop(0, NCH // 2, step, 0)
  drain(NCH - 1, 0)
  compute(NCH - 1, 0)

  pltpu.sync_copy(out_v, out.at[pl.ds(base, EPW)])


_dot_sc = functools.partial(
    pl.kernel,
    out_type=jax.ShapeDtypeStruct((E,), jnp.float32),
    mesh=plsc.VectorSubcoreMesh(
        core_axis_name="c", subcore_axis_name="s",
        num_cores=NC, num_subcores=NS),
    compiler_params=pltpu.CompilerParams(needs_layout_passes=False),
    scratch_types=[
        pltpu.VMEM((EPW,), jnp.int32),
        pltpu.VMEM((EPW,), jnp.int32),
        pltpu.VMEM((2, CH, D), jnp.float32),
        pltpu.VMEM((2, CH, D), jnp.float32),
        pltpu.VMEM((EPW,), jnp.float32),
        pltpu.SemaphoreType.DMA((4,)),
    ],
)(_dot_body)


@jax.jit
def kernel(h, edge_index):
  return _dot_sc(h, edge_index[0], edge_index[1])


# speedup vs baseline: 6.8783x; 1.0025x over previous
import functools

import jax
import jax.numpy as jnp
from jax import lax
from jax.experimental import pallas as pl
from jax.experimental.pallas import tpu as pltpu
from jax.experimental.pallas import tpu_sc as plsc

NC, NS, L = 2, 16, 16
NW = NC * NS
E = 320000
EPW = E // NW
CH = 80
NCH = EPW // CH
D = 128
DW = D // 2


def _dot_body(hb, src, dst, out, idx_s, idx_d, rows_s, rows_d, out_v, sems):
  wid = lax.axis_index("s") * NC + lax.axis_index("c")
  base = wid * EPW

  pltpu.sync_copy(src.at[pl.ds(base, EPW)], idx_s)
  pltpu.sync_copy(dst.at[pl.ds(base, EPW)], idx_d)

  def fire(g, b):
    isl = idx_s.at[pl.ds(g * CH, CH)]
    idl = idx_d.at[pl.ds(g * CH, CH)]
    pltpu.async_copy(hb.at[isl], rows_s.at[b], sems.at[2 * b])
    pltpu.async_copy(hb.at[idl], rows_d.at[b], sems.at[2 * b + 1])

  def drain(g, b):
    isl = idx_s.at[pl.ds(g * CH, CH)]
    idl = idx_d.at[pl.ds(g * CH, CH)]
    pltpu.make_async_copy(hb.at[isl], rows_s.at[b], sems.at[2 * b]).wait()
    pltpu.make_async_copy(hb.at[idl], rows_d.at[b], sems.at[2 * b + 1]).wait()

  lane = lax.iota(jnp.int32, L)
  last = lane == (L - 1)

  def compute(g, b):
    rs = rows_s.at[b]
    rd = rows_d.at[b]

    def edge(e, carry2):
      p = None
      for j in range(DW // L):
        a = plsc.bitcast(rs[e, pl.ds(j * L, L)], jnp.bfloat16)
        bb = plsc.bitcast(rd[e, pl.ds(j * L, L)], jnp.bfloat16)
        t = a * bb
        p = t if p is None else p + t
      lo, hi = plsc.unpack(p, format=plsc.PackFormat.INTERLEAVED)
      tot = plsc.cumsum(lo + hi)
      idx = jnp.full((L,), g * CH + e, jnp.int32)
      plsc.store_scatter(out_v, [idx], tot, mask=last)
      return carry2

    lax.fori_loop(0, CH, edge, 0, unroll=2)

  fire(0, 0)

  def step(gg, carry):
    for b in range(2):
      g = 2 * gg + b

      @pl.when(g + 1 < NCH)
      def _():
        fire(g + 1, 1 - b)

      drain(g, b)
      compute(g, b)
    return carry

  lax.fori_loop(0, NCH // 2, step, 0)
  drain(NCH - 1, 0)
  compute(NCH - 1, 0)

  pltpu.sync_copy(out_v, out.at[pl.ds(base, EPW)])


_dot_sc = functools.partial(
    pl.kernel,
    out_type=jax.ShapeDtypeStruct((E,), jnp.float32),
    mesh=plsc.VectorSubcoreMesh(
        core_axis_name="c", subcore_axis_name="s",
        num_cores=NC, num_subcores=NS),
    compiler_params=pltpu.CompilerParams(needs_layout_passes=False, use_tc_tiling_on_sc=False),
    scratch_types=[
        pltpu.VMEM((EPW,), jnp.int32),
        pltpu.VMEM((EPW,), jnp.int32),
        pltpu.VMEM((2, CH, DW), jnp.int32),
        pltpu.VMEM((2, CH, DW), jnp.int32),
        pltpu.VMEM((EPW,), jnp.float32),
        pltpu.SemaphoreType.DMA((4,)),
    ],
)(_dot_body)


@jax.jit
def kernel(h, edge_index):
  hb16 = h.astype(jnp.bfloat16)
  hb = lax.bitcast_convert_type(hb16.reshape(h.shape[0], DW, 2), jnp.int32)
  return _dot_sc(hb, edge_index[0], edge_index[1])


# speedup vs baseline: 10.8990x; 1.5846x over previous
import functools

import jax
import jax.numpy as jnp
from jax import lax
from jax.experimental import pallas as pl
from jax.experimental.pallas import tpu as pltpu
from jax.experimental.pallas import tpu_sc as plsc

NC, NS, L = 2, 16, 16
NW = NC * NS
E = 320000
EPW = E // NW
CH = 80
NCH = EPW // CH
D = 128
DW = D // 2


def _dot_body(hb, src, dst, out, idx_s, idx_d, rows_s, rows_d, out_v, sems):
  wid = lax.axis_index("s") * NC + lax.axis_index("c")
  base = wid * EPW

  pltpu.sync_copy(src.at[pl.ds(base, EPW)], idx_s)
  pltpu.sync_copy(dst.at[pl.ds(base, EPW)], idx_d)

  def fire(g, b):
    isl = idx_s.at[pl.ds(g * CH, CH)]
    idl = idx_d.at[pl.ds(g * CH, CH)]
    pltpu.async_copy(hb.at[isl], rows_s.at[b], sems.at[2 * b])
    pltpu.async_copy(hb.at[idl], rows_d.at[b], sems.at[2 * b + 1])

  def drain(g, b):
    isl = idx_s.at[pl.ds(g * CH, CH)]
    idl = idx_d.at[pl.ds(g * CH, CH)]
    pltpu.make_async_copy(hb.at[isl], rows_s.at[b], sems.at[2 * b]).wait()
    pltpu.make_async_copy(hb.at[idl], rows_d.at[b], sems.at[2 * b + 1]).wait()

  lane = lax.iota(jnp.int32, L)
  last = lane == (L - 1)

  def compute(g, b):
    rs = rows_s.at[b]
    rd = rows_d.at[b]

    @plsc.parallel_loop(0, CH, step=1, unroll=8)
    def edge(e):
      p = None
      for j in range(DW // L):
        a = plsc.bitcast(rs[e, pl.ds(j * L, L)], jnp.bfloat16)
        bb = plsc.bitcast(rd[e, pl.ds(j * L, L)], jnp.bfloat16)
        t = a * bb
        p = t if p is None else p + t
      lo, hi = plsc.unpack(p, format=plsc.PackFormat.INTERLEAVED)
      tot = plsc.cumsum(lo + hi)
      idx = jnp.full((L,), g * CH + e, jnp.int32)
      plsc.store_scatter(out_v, [idx], tot, mask=last)

  fire(0, 0)

  def step(gg, carry):
    for b in range(2):
      g = 2 * gg + b

      @pl.when(g + 1 < NCH)
      def _():
        fire(g + 1, 1 - b)

      drain(g, b)
      compute(g, b)
    return carry

  lax.fori_loop(0, NCH // 2, step, 0)
  drain(NCH - 1, 0)
  compute(NCH - 1, 0)

  pltpu.sync_copy(out_v, out.at[pl.ds(base, EPW)])


_dot_sc = functools.partial(
    pl.kernel,
    out_type=jax.ShapeDtypeStruct((E,), jnp.float32),
    mesh=plsc.VectorSubcoreMesh(
        core_axis_name="c", subcore_axis_name="s",
        num_cores=NC, num_subcores=NS),
    compiler_params=pltpu.CompilerParams(needs_layout_passes=False, use_tc_tiling_on_sc=False),
    scratch_types=[
        pltpu.VMEM((EPW,), jnp.int32),
        pltpu.VMEM((EPW,), jnp.int32),
        pltpu.VMEM((2, CH, DW), jnp.int32),
        pltpu.VMEM((2, CH, DW), jnp.int32),
        pltpu.VMEM((EPW,), jnp.float32),
        pltpu.SemaphoreType.DMA((4,)),
    ],
)(_dot_body)


@jax.jit
def kernel(h, edge_index):
  hb16 = h.astype(jnp.bfloat16)
  hb = lax.bitcast_convert_type(hb16.reshape(h.shape[0], DW, 2), jnp.int32)
  return _dot_sc(hb, edge_index[0], edge_index[1])


# speedup vs baseline: 12.9522x; 1.1884x over previous
import functools

import jax
import jax.numpy as jnp
from jax import lax
from jax.experimental import pallas as pl
from jax.experimental.pallas import tpu as pltpu
from jax.experimental.pallas import tpu_sc as plsc

NC, NS, L = 2, 16, 16
NW = NC * NS
E = 320000
EPW = E // NW
CH = 80
NCH = EPW // CH
D = 128
DW = D // 2


H = 10000


def _dot_body(hb, src, dst, out, hs, idx_s, idx_d, rows_s, rows_d, out_v, sems):
  sid = lax.axis_index("s")
  wid = sid * NC + lax.axis_index("c")
  base = wid * EPW

  rp = H // NS
  pltpu.sync_copy(hb.at[pl.ds(sid * rp, rp)], hs.at[pl.ds(sid * rp, rp)])
  plsc.subcore_barrier()

  pltpu.sync_copy(src.at[pl.ds(base, EPW)], idx_s)
  pltpu.sync_copy(dst.at[pl.ds(base, EPW)], idx_d)

  def fire(g, b):
    isl = idx_s.at[pl.ds(g * CH, CH)]
    idl = idx_d.at[pl.ds(g * CH, CH)]
    pltpu.async_copy(hs.at[isl], rows_s.at[b], sems.at[2 * b])
    pltpu.async_copy(hs.at[idl], rows_d.at[b], sems.at[2 * b + 1])

  def drain(g, b):
    isl = idx_s.at[pl.ds(g * CH, CH)]
    idl = idx_d.at[pl.ds(g * CH, CH)]
    pltpu.make_async_copy(hs.at[isl], rows_s.at[b], sems.at[2 * b]).wait()
    pltpu.make_async_copy(hs.at[idl], rows_d.at[b], sems.at[2 * b + 1]).wait()

  lane = lax.iota(jnp.int32, L)
  last = lane == (L - 1)

  def compute(g, b):
    rs = rows_s.at[b]
    rd = rows_d.at[b]

    @plsc.parallel_loop(0, CH, step=1, unroll=8)
    def edge(e):
      p = None
      for j in range(DW // L):
        a = plsc.bitcast(rs[e, pl.ds(j * L, L)], jnp.bfloat16)
        bb = plsc.bitcast(rd[e, pl.ds(j * L, L)], jnp.bfloat16)
        t = a * bb
        p = t if p is None else p + t
      lo, hi = plsc.unpack(p, format=plsc.PackFormat.INTERLEAVED)
      tot = plsc.cumsum(lo + hi)
      idx = jnp.full((L,), g * CH + e, jnp.int32)
      plsc.store_scatter(out_v, [idx], tot, mask=last)

  fire(0, 0)

  def step(gg, carry):
    for b in range(2):
      g = 2 * gg + b

      @pl.when(g + 1 < NCH)
      def _():
        fire(g + 1, 1 - b)

      drain(g, b)
      compute(g, b)
    return carry

  lax.fori_loop(0, NCH // 2, step, 0)
  drain(NCH - 1, 0)
  compute(NCH - 1, 0)

  pltpu.sync_copy(out_v, out.at[pl.ds(base, EPW)])


_dot_sc = functools.partial(
    pl.kernel,
    out_type=jax.ShapeDtypeStruct((E,), jnp.float32),
    mesh=plsc.VectorSubcoreMesh(
        core_axis_name="c", subcore_axis_name="s",
        num_cores=NC, num_subcores=NS),
    compiler_params=pltpu.CompilerParams(needs_layout_passes=False, use_tc_tiling_on_sc=False),
    scratch_types=[
        pltpu.VMEM_SHARED((10000, DW), jnp.int32),
        pltpu.VMEM((EPW,), jnp.int32),
        pltpu.VMEM((EPW,), jnp.int32),
        pltpu.VMEM((2, CH, DW), jnp.int32),
        pltpu.VMEM((2, CH, DW), jnp.int32),
        pltpu.VMEM((EPW,), jnp.float32),
        pltpu.SemaphoreType.DMA((4,)),
    ],
)(_dot_body)


@jax.jit
def kernel(h, edge_index):
  hb16 = h.astype(jnp.bfloat16)
  hb = lax.bitcast_convert_type(hb16.reshape(h.shape[0], DW, 2), jnp.int32)
  return _dot_sc(hb, edge_index[0], edge_index[1])


# speedup vs baseline: 17.8917x; 1.3814x over previous
import functools

import jax
import jax.numpy as jnp
from jax import lax
from jax.experimental import pallas as pl
from jax.experimental.pallas import tpu as pltpu
from jax.experimental.pallas import tpu_sc as plsc

NC, NS, L = 2, 16, 16
NW = NC * NS
E = 320000
EPW = E // NW
CH = 80
NCH = EPW // CH
D = 128
DW = D // 2


H = 10000


def _dot_body(hb, eidx, out, hs, idx_s, idx_d, rows_s, rows_d, out_v, sems):
  sid = lax.axis_index("s")
  wid = sid * NC + lax.axis_index("c")
  base = wid * EPW

  rp = H // NS
  pltpu.sync_copy(hb.at[pl.ds(sid * rp, rp)], hs.at[pl.ds(sid * rp, rp)])
  plsc.subcore_barrier()

  pltpu.sync_copy(eidx.at[0, pl.ds(base, EPW)], idx_s)
  pltpu.sync_copy(eidx.at[1, pl.ds(base, EPW)], idx_d)

  def fire(g, b):
    isl = idx_s.at[pl.ds(g * CH, CH)]
    idl = idx_d.at[pl.ds(g * CH, CH)]
    pltpu.async_copy(hs.at[isl], rows_s.at[b], sems.at[2 * b])
    pltpu.async_copy(hs.at[idl], rows_d.at[b], sems.at[2 * b + 1])

  def drain(g, b):
    isl = idx_s.at[pl.ds(g * CH, CH)]
    idl = idx_d.at[pl.ds(g * CH, CH)]
    pltpu.make_async_copy(hs.at[isl], rows_s.at[b], sems.at[2 * b]).wait()
    pltpu.make_async_copy(hs.at[idl], rows_d.at[b], sems.at[2 * b + 1]).wait()

  lane = lax.iota(jnp.int32, L)
  last = lane == (L - 1)

  def compute(g, b):
    rs = rows_s.at[b]
    rd = rows_d.at[b]

    @plsc.parallel_loop(0, CH, step=1, unroll=8)
    def edge(e):
      p = None
      for j in range(DW // L):
        a = plsc.bitcast(rs[e, pl.ds(j * L, L)], jnp.bfloat16)
        bb = plsc.bitcast(rd[e, pl.ds(j * L, L)], jnp.bfloat16)
        t = a * bb
        p = t if p is None else p + t
      lo, hi = plsc.unpack(p, format=plsc.PackFormat.INTERLEAVED)
      tot = plsc.cumsum(lo + hi)
      idx = jnp.full((L,), g * CH + e, jnp.int32)
      plsc.store_scatter(out_v, [idx], tot, mask=last)

  fire(0, 0)

  def step(gg, carry):
    for b in range(2):
      g = 2 * gg + b

      @pl.when(g + 1 < NCH)
      def _():
        fire(g + 1, 1 - b)

      drain(g, b)
      compute(g, b)
    return carry

  lax.fori_loop(0, NCH // 2, step, 0)
  drain(NCH - 1, 0)
  compute(NCH - 1, 0)

  pltpu.sync_copy(out_v, out.at[pl.ds(base, EPW)])


_dot_sc = functools.partial(
    pl.kernel,
    out_type=jax.ShapeDtypeStruct((E,), jnp.float32),
    mesh=plsc.VectorSubcoreMesh(
        core_axis_name="c", subcore_axis_name="s",
        num_cores=NC, num_subcores=NS),
    compiler_params=pltpu.CompilerParams(needs_layout_passes=False, use_tc_tiling_on_sc=False),
    scratch_types=[
        pltpu.VMEM_SHARED((10000, DW), jnp.int32),
        pltpu.VMEM((EPW,), jnp.int32),
        pltpu.VMEM((EPW,), jnp.int32),
        pltpu.VMEM((2, CH, DW), jnp.int32),
        pltpu.VMEM((2, CH, DW), jnp.int32),
        pltpu.VMEM((EPW,), jnp.float32),
        pltpu.SemaphoreType.DMA((4,)),
    ],
)(_dot_body)


@jax.jit
def kernel(h, edge_index):
  lo = lax.bitcast_convert_type(h[:, :DW].astype(jnp.bfloat16), jnp.uint16)
  hi = lax.bitcast_convert_type(h[:, DW:].astype(jnp.bfloat16), jnp.uint16)
  hb = lax.bitcast_convert_type(
      lo.astype(jnp.uint32) | (hi.astype(jnp.uint32) << 16), jnp.int32)
  return _dot_sc(hb, edge_index)
